# baseline (speedup 1.0000x reference)
import math
import functools

import jax
import jax.numpy as jnp
from jax.experimental import pallas as pl
from jax.experimental.pallas import tpu as pltpu

N = 10000
E = 160000
H = 256
C = 64

_BLK = 256
_NPAD = 10240


def _dense_body(x_ref, w1_ref, b1_ref, w2_ref, b2_ref, lng_ref, lnb_ref,
                wl_ref, bl_ref, wr_ref, br_ref, q_ref, k_ref):
    x = x_ref[...]
    mid = jax.nn.gelu(
        jnp.dot(x, w1_ref[...], preferred_element_type=jnp.float32)
        + b1_ref[...])
    h = jnp.dot(mid, w2_ref[...], preferred_element_type=jnp.float32) + b2_ref[...]
    mu = jnp.mean(h, axis=-1, keepdims=True)
    var = jnp.mean((h - mu) ** 2, axis=-1, keepdims=True)
    h = (h - mu) * jax.lax.rsqrt(var + 1e-5) * lng_ref[...] + lnb_ref[...]
    q_ref[...] = jnp.dot(h, wl_ref[...], preferred_element_type=jnp.float32) + bl_ref[...]
    k_ref[...] = jnp.dot(h, wr_ref[...], preferred_element_type=jnp.float32) + br_ref[...]


@functools.partial(jax.jit, static_argnames=())
def _dense_stage(emb_pad, W1, b1, W2, b2, ln_g, ln_b, Wl, bl, Wr, br):
    grid = _NPAD // _BLK
    row_spec = pl.BlockSpec((_BLK, H), lambda i: (i, 0))
    full = lambda shape: pl.BlockSpec(shape, lambda i: tuple(0 for _ in shape))
    q, k = pl.pallas_call(
        _dense_body,
        grid=(grid,),
        in_specs=[
            row_spec,
            full((H, 4 * H)), full((4 * H,)),
            full((4 * H, H)), full((H,)),
            full((H,)), full((H,)),
            full((H, H)), full((H,)),
            full((H, H)), full((H,)),
        ],
        out_specs=[row_spec, row_spec],
        out_shape=[
            jax.ShapeDtypeStruct((_NPAD, H), jnp.float32),
            jax.ShapeDtypeStruct((_NPAD, H), jnp.float32),
        ],
    )(emb_pad, W1, b1, W2, b2, ln_g, ln_b, Wl, bl, Wr, br)
    return q, k


def kernel(embedding, scores, W1, b1, W2, b2, ln_g, ln_b, Wl, bl, Wr, br,
           We, be, emb_table, edge_index, type_int):
    emb_pad = jnp.pad(embedding, ((0, _NPAD - N), (0, 0)))
    attn_q, attn_k1 = _dense_stage(emb_pad, W1, b1, W2, b2, ln_g, ln_b,
                                   Wl, bl, Wr, br)
    attn_q = attn_q[:N]
    attn_k1 = attn_k1[:N]

    src = edge_index[0]
    dst = edge_index[1]
    edge_type_feat = jax.nn.gelu(jnp.take(emb_table, type_int, axis=0))
    attn_k2 = edge_type_feat @ We + be
    attn_k = jnp.take(attn_k1, src, axis=0) + attn_k2
    alpha = jnp.sum(jnp.take(attn_q, dst, axis=0) * attn_k, axis=-1) / math.sqrt(float(H))
    m = jax.ops.segment_max(alpha, dst, num_segments=N)
    ex = jnp.exp(alpha - jnp.take(m, dst))
    denom = jax.ops.segment_sum(ex, dst, num_segments=N)
    a = ex / jnp.take(denom, dst)
    a_scores = jnp.take(scores, src, axis=0) * a[:, None]
    updated_scores = jax.ops.segment_sum(a_scores, dst, num_segments=N)
    return updated_scores
